# baseline (speedup 1.0000x reference)
import jax
import jax.numpy as jnp
from jax.experimental import pallas as pl
from jax.experimental.pallas import tpu as pltpu

SS = 8
BW = 2048
NBUF = 2


def _body(x_hbm, st_hbm, len_ref, inb, outb, insem, outsem):
    S, B, D = x_hbm.shape
    ns = S // SS
    nstep = (B // BW) * ns
    i = pl.program_id(0)

    def in_copy(step, slot):
        bb = step // ns
        ss = step % ns
        return pltpu.make_async_copy(
            x_hbm.at[pl.ds(ss * SS, SS), pl.ds(bb * BW, BW), :],
            inb.at[slot],
            insem.at[slot],
        )

    def out_copy(step, slot):
        bb = step // ns
        ss = step % ns
        return pltpu.make_async_copy(
            outb.at[slot],
            st_hbm.at[pl.ds(bb * BW, BW), pl.ds(ss * SS, SS), :],
            outsem.at[slot],
        )

    @pl.when(i == 0)
    def _prime():
        in_copy(0, 0).start()
        in_copy(1, 1).start()

    slot = jax.lax.rem(i, NBUF)
    in_copy(i, slot).wait()

    @pl.when(i >= NBUF)
    def _free_out():
        out_copy(i - NBUF, slot).wait()

    x = inb[slot]
    outb[slot] = jnp.swapaxes(x, 0, 1)
    out_copy(i, slot).start()

    @pl.when(i + NBUF < nstep)
    def _next_in():
        in_copy(i + NBUF, slot).start()

    rs = jnp.sum(x, axis=2)
    cnt = jnp.sum((rs != 0.0).astype(jnp.int32), axis=0)
    bb = i // ns

    @pl.when(i == 0)
    def _init_len():
        len_ref[...] = jnp.zeros_like(len_ref)

    len_ref[0, pl.ds(bb * BW, BW)] += cnt

    @pl.when(i == nstep - 1)
    def _drain():
        out_copy(i - 1, jax.lax.rem(i - 1, NBUF)).wait()
        out_copy(i, slot).wait()


def kernel(batch):
    S, B, D = batch.shape
    nstep = (B // BW) * (S // SS)
    states, lengths2d = pl.pallas_call(
        _body,
        grid=(nstep,),
        in_specs=[pl.BlockSpec(memory_space=pl.ANY)],
        out_specs=[
            pl.BlockSpec(memory_space=pl.ANY),
            pl.BlockSpec((1, B), lambda i: (0, 0)),
        ],
        out_shape=[
            jax.ShapeDtypeStruct((B, S, D), jnp.float32),
            jax.ShapeDtypeStruct((1, B), jnp.int32),
        ],
        scratch_shapes=[
            pltpu.VMEM((NBUF, SS, BW, D), jnp.float32),
            pltpu.VMEM((NBUF, BW, SS, D), jnp.float32),
            pltpu.SemaphoreType.DMA((NBUF,)),
            pltpu.SemaphoreType.DMA((NBUF,)),
        ],
    )(batch)
    return states, lengths2d.reshape(B)


# speedup vs baseline: 6.3408x; 6.3408x over previous
import jax
import jax.numpy as jnp
from jax.experimental import pallas as pl
from jax.experimental.pallas import tpu as pltpu


def _body(x_ref, out_ref, len_ref):
    s = pl.program_id(0)
    x = x_ref[...]
    out_ref[...] = x
    rs = jnp.sum(x, axis=1)
    cnt = jnp.sum((rs != 0.0).astype(jnp.int32), axis=0)

    @pl.when(s == 0)
    def _init():
        len_ref[...] = jnp.zeros_like(len_ref)

    len_ref[...] += cnt[None, :]


def kernel(batch):
    S, B, D = batch.shape
    x2 = jnp.transpose(batch, (0, 2, 1))
    sS = 8
    out2, lengths2d = pl.pallas_call(
        _body,
        grid=(S // sS,),
        in_specs=[pl.BlockSpec((sS, D, B), lambda s: (s, 0, 0))],
        out_specs=[
            pl.BlockSpec((sS, D, B), lambda s: (s, 0, 0)),
            pl.BlockSpec((1, B), lambda s: (0, 0)),
        ],
        out_shape=[
            jax.ShapeDtypeStruct((S, D, B), jnp.float32),
            jax.ShapeDtypeStruct((1, B), jnp.int32),
        ],
        compiler_params=pltpu.CompilerParams(
            dimension_semantics=("arbitrary",),
        ),
    )(x2)
    states = jnp.transpose(out2, (2, 0, 1))
    return states, lengths2d.reshape(B)
